# baseline (speedup 1.0000x reference)
import dataclasses
import functools

import jax
import jax.numpy as jnp
from jax import lax
from jax.experimental import pallas as pl
from jax.experimental.pallas import tpu as pltpu
from jax.experimental.pallas import tpu_sc as plsc

VOCAB_PAD = 128
NC = 2
NS = 16
NW = NC * NS
LANES = 16
CHUNK = 128


def _sc_hist(idx_hbm, cnt_hbm, idx_v0, idx_v1, cnt_v0, cnt_v1,
             isem0, isem1, osem0, osem1, *, batch, length):
    wid = lax.axis_index("s") * NC + lax.axis_index("c")
    rows_w = batch // NW
    base_w = wid * rows_w
    nchunks = rows_w // CHUNK
    ones = jnp.ones((LANES,), jnp.float32)
    zeros = jnp.zeros((LANES,), jnp.float32)
    lane = lax.broadcasted_iota(jnp.int32, (LANES,), 0)
    tail_fresh = lane >= (LANES - length % LANES)

    ibufs, cbufs = [idx_v0, idx_v1], [cnt_v0, cnt_v1]
    isems, osems = [isem0, isem1], [osem0, osem1]

    def start_in(k):
        base = base_w + k * CHUNK
        return pltpu.async_copy(
            idx_hbm.at[pl.ds(pl.multiple_of(base, 8), CHUNK)],
            ibufs[k % 2], isems[k % 2])

    def start_out(k):
        base = base_w + k * CHUNK
        return pltpu.async_copy(
            cbufs[k % 2],
            cnt_hbm.at[pl.ds(pl.multiple_of(base * VOCAB_PAD, 8),
                             CHUNK * VOCAB_PAD)],
            osems[k % 2])

    in_copies = [start_in(0)]
    out_copies = []
    for k in range(nchunks):
        if k + 1 < nchunks:
            in_copies.append(start_in(k + 1))
        in_copies[k].wait()
        if k >= 2:
            out_copies[k - 2].wait()
        idx_v, cnt_v = ibufs[k % 2], cbufs[k % 2]

        @plsc.parallel_loop(0, CHUNK, unroll=2)
        def _row(r):
            rvec = jnp.full((LANES,), r, jnp.int32)
            rcnt = cnt_v.at[pl.ds(pl.multiple_of(r * VOCAB_PAD, 8), VOCAB_PAD)]
            for j in range(0, VOCAB_PAD, LANES):
                rcnt[pl.ds(j, LANES)] = zeros
            for c in range(0, length - LANES, LANES):
                vals = plsc.load_gather(idx_v, [rvec, lane + c])
                plsc.addupdate_scatter(rcnt, [vals], ones)
            tail = plsc.load_gather(idx_v, [rvec, lane + (length - LANES)])
            plsc.addupdate_scatter(rcnt, [tail], ones, mask=tail_fresh)

        out_copies.append(start_out(k))
    for k in (nchunks - 2, nchunks - 1):
        out_copies[k].wait()


def _matmul_kernel(cnt_ref, tab2_ref, out_ref, *, length):
    c = cnt_ref[...].astype(jnp.bfloat16)
    c2 = jnp.concatenate([c, c], axis=1)
    acc = lax.dot_general(c2, tab2_ref[...], (((1,), (0,)), ((), ())),
                          preferred_element_type=jnp.float32)
    out_ref[...] = acc * (1.0 / length)


def _sc_counts(idx_part, length, cp, mesh):
    Bp = idx_part.shape[0]
    counts = pl.kernel(
        functools.partial(_sc_hist, batch=Bp, length=length),
        out_type=jax.ShapeDtypeStruct((Bp * VOCAB_PAD,), jnp.float32),
        mesh=mesh,
        scratch_types=[
            pltpu.VMEM((CHUNK, length), jnp.int32),
            pltpu.VMEM((CHUNK, length), jnp.int32),
            pltpu.VMEM((CHUNK * VOCAB_PAD,), jnp.float32),
            pltpu.VMEM((CHUNK * VOCAB_PAD,), jnp.float32),
            pltpu.SemaphoreType.DMA,
            pltpu.SemaphoreType.DMA,
            pltpu.SemaphoreType.DMA,
            pltpu.SemaphoreType.DMA,
        ],
        compiler_params=cp,
    )(idx_part)
    return counts.reshape(Bp, VOCAB_PAD)


def _matmul(counts, tab2, length):
    Bp = counts.shape[0]
    BM = 2048
    return pl.pallas_call(
        functools.partial(_matmul_kernel, length=length),
        grid=(Bp // BM,),
        in_specs=[
            pl.BlockSpec((BM, VOCAB_PAD), lambda i: (i, 0)),
            pl.BlockSpec((2 * VOCAB_PAD, D_GLOBAL), lambda i: (0, 0)),
        ],
        out_specs=pl.BlockSpec((BM, D_GLOBAL), lambda i: (i, 0)),
        out_shape=jax.ShapeDtypeStruct((Bp, D_GLOBAL), jnp.float32),
    )(counts, tab2)


D_GLOBAL = 128


def kernel(indices, character_embedding):
    B, L = indices.shape
    V, D = character_embedding.shape
    idx = indices.astype(jnp.int32)
    tab = jnp.zeros((VOCAB_PAD, D), jnp.float32).at[:V, :].set(character_embedding)

    cp = pltpu.CompilerParams(use_tc_tiling_on_sc=True)
    if "needs_layout_passes" in pltpu.CompilerParams.__dataclass_fields__:
        cp = dataclasses.replace(cp, needs_layout_passes=False)
    mesh = plsc.VectorSubcoreMesh(core_axis_name="c", subcore_axis_name="s")

    tbits = lax.bitcast_convert_type(tab, jnp.uint32)
    hi_f = lax.bitcast_convert_type(tbits & jnp.uint32(0xFFFF0000),
                                    jnp.float32)
    tab_hi = hi_f.astype(jnp.bfloat16)
    tab_lo = (tab - hi_f).astype(jnp.bfloat16)
    tab2 = jnp.concatenate([tab_hi, tab_lo], axis=0)

    h = B // 2
    counts0 = _sc_counts(idx[:h], L, cp, mesh)
    counts1 = _sc_counts(idx[h:], L, cp, mesh)
    out0 = _matmul(counts0, tab2, L)
    out1 = _matmul(counts1, tab2, L)
    return jnp.concatenate([out0, out1], axis=0)


# speedup vs baseline: 1.4022x; 1.4022x over previous
import dataclasses
import functools

import jax
import jax.numpy as jnp
from jax import lax
from jax.experimental import pallas as pl
from jax.experimental.pallas import tpu as pltpu
from jax.experimental.pallas import tpu_sc as plsc

VOCAB_PAD = 128
NC = 2
NS = 16
NW = NC * NS
LANES = 16
CHUNK = 128


def _sc_hist(idx_hbm, cnt_hbm, idx_v0, idx_v1, cnt_v0, cnt_v1,
             isem0, isem1, osem0, osem1, *, batch, length):
    wid = lax.axis_index("s") * NC + lax.axis_index("c")
    rows_w = batch // NW
    base_w = wid * rows_w
    nchunks = rows_w // CHUNK
    ones = jnp.ones((LANES,), jnp.float32)
    zeros = jnp.zeros((LANES,), jnp.float32)
    lane = lax.broadcasted_iota(jnp.int32, (LANES,), 0)
    tail_fresh = lane >= (LANES - length % LANES)

    ibufs, cbufs = [idx_v0, idx_v1], [cnt_v0, cnt_v1]
    isems, osems = [isem0, isem1], [osem0, osem1]

    def start_in(k):
        base = base_w + k * CHUNK
        return pltpu.async_copy(
            idx_hbm.at[pl.ds(pl.multiple_of(base, 8), CHUNK)],
            ibufs[k % 2], isems[k % 2])

    def start_out(k):
        base = base_w + k * CHUNK
        return pltpu.async_copy(
            cbufs[k % 2],
            cnt_hbm.at[pl.ds(pl.multiple_of(base * VOCAB_PAD, 8),
                             CHUNK * VOCAB_PAD)],
            osems[k % 2])

    in_copies = [start_in(0)]
    out_copies = []
    for k in range(nchunks):
        if k + 1 < nchunks:
            in_copies.append(start_in(k + 1))
        in_copies[k].wait()
        if k >= 2:
            out_copies[k - 2].wait()
        idx_v, cnt_v = ibufs[k % 2], cbufs[k % 2]

        @plsc.parallel_loop(0, CHUNK, unroll=2)
        def _row(r):
            rvec = jnp.full((LANES,), r, jnp.int32)
            rcnt = cnt_v.at[pl.ds(pl.multiple_of(r * VOCAB_PAD, 8), VOCAB_PAD)]
            for j in range(0, VOCAB_PAD, LANES):
                rcnt[pl.ds(j, LANES)] = zeros
            for c in range(0, length - LANES, LANES):
                vals = plsc.load_gather(idx_v, [rvec, lane + c])
                plsc.addupdate_scatter(rcnt, [vals], ones)
            tail = plsc.load_gather(idx_v, [rvec, lane + (length - LANES)])
            plsc.addupdate_scatter(rcnt, [tail], ones, mask=tail_fresh)

        out_copies.append(start_out(k))
    for k in (nchunks - 2, nchunks - 1):
        out_copies[k].wait()


def _matmul_kernel(cnt_ref, tab2_ref, out_ref, *, length):
    c = cnt_ref[...].astype(jnp.bfloat16)
    c2 = jnp.concatenate([c, c], axis=1)
    acc = lax.dot_general(c2, tab2_ref[...], (((1,), (0,)), ((), ())),
                          preferred_element_type=jnp.float32)
    out_ref[...] = acc * (1.0 / length)


def kernel(indices, character_embedding):
    B, L = indices.shape
    V, D = character_embedding.shape
    idx = indices.astype(jnp.int32)
    tab = jnp.zeros((VOCAB_PAD, D), jnp.float32).at[:V, :].set(character_embedding)

    cp = pltpu.CompilerParams(use_tc_tiling_on_sc=True)
    if "needs_layout_passes" in pltpu.CompilerParams.__dataclass_fields__:
        cp = dataclasses.replace(cp, needs_layout_passes=False)
    mesh = plsc.VectorSubcoreMesh(core_axis_name="c", subcore_axis_name="s")
    counts = pl.kernel(
        functools.partial(_sc_hist, batch=B, length=L),
        out_type=jax.ShapeDtypeStruct((B * VOCAB_PAD,), jnp.float32),
        mesh=mesh,
        scratch_types=[
            pltpu.VMEM((CHUNK, L), jnp.int32),
            pltpu.VMEM((CHUNK, L), jnp.int32),
            pltpu.VMEM((CHUNK * VOCAB_PAD,), jnp.float32),
            pltpu.VMEM((CHUNK * VOCAB_PAD,), jnp.float32),
            pltpu.SemaphoreType.DMA,
            pltpu.SemaphoreType.DMA,
            pltpu.SemaphoreType.DMA,
            pltpu.SemaphoreType.DMA,
        ],
        compiler_params=cp,
    )(idx)
    counts = counts.reshape(B, VOCAB_PAD)

    tbits = lax.bitcast_convert_type(tab, jnp.uint32)
    hi_f = lax.bitcast_convert_type(tbits & jnp.uint32(0xFFFF0000),
                                    jnp.float32)
    tab_hi = hi_f.astype(jnp.bfloat16)
    tab_lo = (tab - hi_f).astype(jnp.bfloat16)
    tab2 = jnp.concatenate([tab_hi, tab_lo], axis=0)
    BM = 4096
    return pl.pallas_call(
        functools.partial(_matmul_kernel, length=L),
        grid=(B // BM,),
        in_specs=[
            pl.BlockSpec((BM, VOCAB_PAD), lambda i: (i, 0)),
            pl.BlockSpec((2 * VOCAB_PAD, D), lambda i: (0, 0)),
        ],
        out_specs=pl.BlockSpec((BM, D), lambda i: (i, 0)),
        out_shape=jax.ShapeDtypeStruct((B, D), jnp.float32),
    )(counts, tab2)


# speedup vs baseline: 1.4363x; 1.0244x over previous
import dataclasses
import functools

import jax
import jax.numpy as jnp
from jax import lax
from jax.experimental import pallas as pl
from jax.experimental.pallas import tpu as pltpu
from jax.experimental.pallas import tpu_sc as plsc

VOCAB_PAD = 128
NC = 2
NS = 16
NW = NC * NS
LANES = 16
CHUNK = 128


def _sc_hist(idx_hbm, cnt_hbm, idx_v0, idx_v1, cnt_v0, cnt_v1,
             isem0, isem1, osem0, osem1, *, batch, length):
    wid = lax.axis_index("s") * NC + lax.axis_index("c")
    rows_w = batch // NW
    base_w = wid * rows_w
    nchunks = rows_w // CHUNK
    ones = jnp.ones((LANES,), jnp.float32)
    zeros = jnp.zeros((LANES,), jnp.float32)
    lane = lax.broadcasted_iota(jnp.int32, (LANES,), 0)
    tail_fresh = lane >= (LANES - length % LANES)

    ibufs, cbufs = [idx_v0, idx_v1], [cnt_v0, cnt_v1]
    isems, osems = [isem0, isem1], [osem0, osem1]

    def start_in(k):
        base = base_w + k * CHUNK
        return pltpu.async_copy(
            idx_hbm.at[pl.ds(pl.multiple_of(base, 8), CHUNK)],
            ibufs[k % 2], isems[k % 2])

    def start_out(k):
        base = base_w + k * CHUNK
        return pltpu.async_copy(
            cbufs[k % 2],
            cnt_hbm.at[pl.ds(pl.multiple_of(base * VOCAB_PAD, 8),
                             CHUNK * VOCAB_PAD)],
            osems[k % 2])

    in_copies = [start_in(0)]
    out_copies = []
    for k in range(nchunks):
        if k + 1 < nchunks:
            in_copies.append(start_in(k + 1))
        in_copies[k].wait()
        if k >= 2:
            out_copies[k - 2].wait()
        idx_v, cnt_v = ibufs[k % 2], cbufs[k % 2]

        @plsc.parallel_loop(0, CHUNK, unroll=2)
        def _row(r):
            rvec = jnp.full((LANES,), r, jnp.int32)
            rcnt = cnt_v.at[pl.ds(pl.multiple_of(r * VOCAB_PAD, 8), VOCAB_PAD)]
            for j in range(0, VOCAB_PAD, LANES):
                rcnt[pl.ds(j, LANES)] = zeros
            for c in range(0, length - LANES, LANES):
                vals = plsc.load_gather(idx_v, [rvec, lane + c])
                plsc.addupdate_scatter(rcnt, [vals], ones)
            tail = plsc.load_gather(idx_v, [rvec, lane + (length - LANES)])
            plsc.addupdate_scatter(rcnt, [tail], ones, mask=tail_fresh)

        out_copies.append(start_out(k))
    for k in (nchunks - 2, nchunks - 1):
        out_copies[k].wait()


def _matmul_kernel(cnt_ref, tab2_ref, out_ref, *, length):
    c = cnt_ref[...].astype(jnp.bfloat16)
    c2 = jnp.concatenate([c, c], axis=1)
    acc = lax.dot_general(c2, tab2_ref[...], (((1,), (0,)), ((), ())),
                          preferred_element_type=jnp.float32)
    out_ref[...] = acc * (1.0 / length)


def kernel(indices, character_embedding):
    B, L = indices.shape
    V, D = character_embedding.shape
    idx = indices.astype(jnp.int32)
    tab = jnp.zeros((VOCAB_PAD, D), jnp.float32).at[:V, :].set(character_embedding)

    cp = pltpu.CompilerParams(use_tc_tiling_on_sc=True)
    if "needs_layout_passes" in pltpu.CompilerParams.__dataclass_fields__:
        cp = dataclasses.replace(cp, needs_layout_passes=False)
    mesh = plsc.VectorSubcoreMesh(core_axis_name="c", subcore_axis_name="s")
    counts = pl.kernel(
        functools.partial(_sc_hist, batch=B, length=L),
        out_type=jax.ShapeDtypeStruct((B * VOCAB_PAD,), jnp.float32),
        mesh=mesh,
        scratch_types=[
            pltpu.VMEM((CHUNK, L), jnp.int32),
            pltpu.VMEM((CHUNK, L), jnp.int32),
            pltpu.VMEM((CHUNK * VOCAB_PAD,), jnp.float32),
            pltpu.VMEM((CHUNK * VOCAB_PAD,), jnp.float32),
            pltpu.SemaphoreType.DMA,
            pltpu.SemaphoreType.DMA,
            pltpu.SemaphoreType.DMA,
            pltpu.SemaphoreType.DMA,
        ],
        compiler_params=cp,
    )(idx)
    counts = counts.reshape(B, VOCAB_PAD)

    tbits = lax.bitcast_convert_type(tab, jnp.uint32)
    hi_f = lax.bitcast_convert_type(tbits & jnp.uint32(0xFFFF0000),
                                    jnp.float32)
    tab_hi = hi_f.astype(jnp.bfloat16)
    tab_lo = (tab - hi_f).astype(jnp.bfloat16)
    tab2 = jnp.concatenate([tab_hi, tab_lo], axis=0)
    BM = 8192
    return pl.pallas_call(
        functools.partial(_matmul_kernel, length=L),
        grid=(B // BM,),
        in_specs=[
            pl.BlockSpec((BM, VOCAB_PAD), lambda i: (i, 0)),
            pl.BlockSpec((2 * VOCAB_PAD, D), lambda i: (0, 0)),
        ],
        out_specs=pl.BlockSpec((BM, D), lambda i: (i, 0)),
        out_shape=jax.ShapeDtypeStruct((B, D), jnp.float32),
    )(counts, tab2)
